# baseline (speedup 1.0000x reference)
import jax
import jax.numpy as jnp
from jax import lax
from jax.experimental import pallas as pl
from jax.experimental.pallas import tpu as pltpu
from jax.experimental.pallas import tpu_sc as plsc

_L = 16
_NW = 32
_E = 32 * 12288
_PER_W = _E // _NW
_CHUNK = 2048
_NCHUNK = _PER_W // _CHUNK
_GROUPS = _CHUNK // _L
_NBUF = 2


def _tile_body(oe_hbm, et_hbm, lab_hbm, out_hbm,
               oe_v, et_v, lab_v, res_v, sems):
    wid = lax.axis_index("s") * 2 + lax.axis_index("c")
    base = wid * _PER_W

    def start(ci, par):
        f0 = base + ci * _CHUNK
        cps = []
        h = _CHUNK // 2
        for c in range(5):
            for k in range(2):
                cps.append(pltpu.async_copy(
                    oe_hbm.at[pl.ds(c * _E + f0 + k * h, h)],
                    oe_v.at[par, c, pl.ds(k * h, h)], sems.at[par]))
                cps.append(pltpu.async_copy(
                    et_hbm.at[pl.ds(c * _E + f0 + k * h, h)],
                    et_v.at[par, c, pl.ds(k * h, h)], sems.at[par]))
        cps.append(pltpu.async_copy(
            lab_hbm.at[pl.ds(f0, _CHUNK)], lab_v.at[par], sems.at[par]))
        return cps

    def compute(par, carry):
        def group(g, carry):
            acc, cnt = carry
            s = g * _L
            dxo = oe_v[par, 0, pl.ds(s, _L)]
            dyo = oe_v[par, 1, pl.ds(s, _L)]
            dlo = oe_v[par, 2, pl.ds(s, _L)]
            dso = oe_v[par, 3, pl.ds(s, _L)]
            to = oe_v[par, 4, pl.ds(s, _L)]
            dxt = et_v[par, 0, pl.ds(s, _L)]
            dyt = et_v[par, 1, pl.ds(s, _L)]
            dlt = et_v[par, 2, pl.ds(s, _L)]
            dst = et_v[par, 3, pl.ds(s, _L)]
            tt = et_v[par, 4, pl.ds(s, _L)]
            lab = lab_v[par, pl.ds(s, _L)]

            r_o = 1.0 / (1.0 + to * to)
            r_t = 1.0 / (1.0 + tt * tt)
            rot = r_o * r_t
            ct = 1.0 + to * tt
            st = to - tt
            c2 = ct * ct * rot
            s2 = st * st * rot
            e_lo = jnp.exp(-2.0 * dlo)
            e_so = jnp.exp(-2.0 * dso)
            e_lt = jnp.exp(2.0 * dlt)
            e_st = jnp.exp(2.0 * dst)
            trace = c2 * (e_lt * e_lo + e_st * e_so) \
                  + s2 * (e_lt * e_so + e_st * e_lo)
            u = dxo - dxt
            v = dyo - dyt
            a = u + to * v
            b = v - to * u
            dist = 4.0 * (a * a * e_lo + b * b * e_so) * r_o
            det2 = (dlo - dlt) + (dso - dst)
            kld = (trace + dist) * 0.5 + det2 - 1.0
            pos = lab == 1
            acc = acc + jnp.where(pos, kld, 0.0)
            cnt = cnt + jnp.where(pos, 1.0, 0.0)
            return acc, cnt

        return lax.fori_loop(0, _GROUPS, group, carry)

    zero = jnp.zeros((_L,), jnp.float32)
    carry = (zero, zero)
    pending = [start(ci, ci) for ci in range(min(_NBUF - 1, _NCHUNK))]
    for ci in range(_NCHUNK):
        nci = ci + _NBUF - 1
        if nci < _NCHUNK:
            pending.append(start(nci, nci % _NBUF))
        for h in pending.pop(0):
            h.wait()
        carry = compute(ci % _NBUF, carry)
    acc, cnt = carry
    res_v[0] = acc
    res_v[1] = cnt
    pltpu.sync_copy(res_v, out_hbm.at[wid])


@jax.jit
def _loss(out_ellipse, labels, ellipse_targets):
    def planes_flat(x):
        t = jnp.transpose(x, (2, 0, 1)).reshape(5, 4, 8, 96, 128)
        return jnp.transpose(t, (0, 1, 3, 2, 4)).reshape(-1)

    oe_flat = planes_flat(out_ellipse)
    et_flat = planes_flat(ellipse_targets)
    lab_flat = jnp.transpose(
        labels.reshape(4, 8, 96, 128), (0, 2, 1, 3)).reshape(-1)

    mesh = plsc.VectorSubcoreMesh(core_axis_name="c", subcore_axis_name="s")
    parts = pl.kernel(
        _tile_body,
        mesh=mesh,
        compiler_params=pltpu.CompilerParams(
            needs_layout_passes=False, use_tc_tiling_on_sc=False),
        out_type=jax.ShapeDtypeStruct((_NW, 2, _L), jnp.float32),
        scratch_types=[
            pltpu.VMEM((_NBUF, 5, _CHUNK), jnp.float32),
            pltpu.VMEM((_NBUF, 5, _CHUNK), jnp.float32),
            pltpu.VMEM((_NBUF, _CHUNK), jnp.int32),
            pltpu.VMEM((2, _L), jnp.float32),
            pltpu.SemaphoreType.DMA((_NBUF,)),
        ],
    )(oe_flat, et_flat, lab_flat)
    total = jnp.sum(parts[:, 0, :])
    npos = jnp.sum(parts[:, 1, :])
    return total / jnp.maximum(npos, 1.0)


def kernel(out_ellipse, labels, ellipse_targets, anchors):
    return _loss(out_ellipse, labels, ellipse_targets)


# speedup vs baseline: 1.0450x; 1.0450x over previous
import jax
import jax.numpy as jnp
from jax import lax
from jax.experimental import pallas as pl
from jax.experimental.pallas import tpu as pltpu
from jax.experimental.pallas import tpu_sc as plsc

_L = 16
_NW = 32
_E = 32 * 12288
_PER_W = _E // _NW
_CHUNK = 2048
_NCHUNK = _PER_W // _CHUNK
_GROUPS = _CHUNK // _L
_NBUF = 2


def _tile_body(oe_hbm, et_hbm, lab_hbm, out_hbm,
               oe_v, et_v, lab_v, res_v, sems):
    wid = lax.axis_index("s") * 2 + lax.axis_index("c")
    base = wid * _PER_W

    def start(ci, par):
        f0 = base + ci * _CHUNK
        for c in range(5):
            pltpu.async_copy(
                oe_hbm.at[pl.ds(c * _E + f0, _CHUNK)],
                oe_v.at[par, pl.ds(c * _CHUNK, _CHUNK)], sems.at[par])
            pltpu.async_copy(
                et_hbm.at[pl.ds(c * _E + f0, _CHUNK)],
                et_v.at[par, pl.ds(c * _CHUNK, _CHUNK)], sems.at[par])
        pltpu.async_copy(
            lab_hbm.at[pl.ds(f0, _CHUNK)], lab_v.at[par], sems.at[par])

    def drain(par):
        pltpu.make_async_copy(
            oe_hbm.at[pl.ds(0, 5 * _CHUNK)], oe_v.at[par],
            sems.at[par]).wait()
        pltpu.make_async_copy(
            et_hbm.at[pl.ds(0, 5 * _CHUNK)], et_v.at[par],
            sems.at[par]).wait()
        pltpu.make_async_copy(
            lab_hbm.at[pl.ds(0, _CHUNK)], lab_v.at[par],
            sems.at[par]).wait()

    def compute(par, carry):
        def group(g, carry):
            acc, cnt = carry
            s = g * _L
            dxo = oe_v[par, pl.ds(s, _L)]
            dyo = oe_v[par, pl.ds(_CHUNK + s, _L)]
            dlo = oe_v[par, pl.ds(2 * _CHUNK + s, _L)]
            dso = oe_v[par, pl.ds(3 * _CHUNK + s, _L)]
            to = oe_v[par, pl.ds(4 * _CHUNK + s, _L)]
            dxt = et_v[par, pl.ds(s, _L)]
            dyt = et_v[par, pl.ds(_CHUNK + s, _L)]
            dlt = et_v[par, pl.ds(2 * _CHUNK + s, _L)]
            dst = et_v[par, pl.ds(3 * _CHUNK + s, _L)]
            tt = et_v[par, pl.ds(4 * _CHUNK + s, _L)]
            lab = lab_v[par, pl.ds(s, _L)]

            r_o = 1.0 / (1.0 + to * to)
            r_t = 1.0 / (1.0 + tt * tt)
            rot = r_o * r_t
            ct = 1.0 + to * tt
            st = to - tt
            c2 = ct * ct * rot
            s2 = st * st * rot
            e_lo = jnp.exp(-2.0 * dlo)
            e_so = jnp.exp(-2.0 * dso)
            e_lt = jnp.exp(2.0 * dlt)
            e_st = jnp.exp(2.0 * dst)
            trace = c2 * (e_lt * e_lo + e_st * e_so) \
                  + s2 * (e_lt * e_so + e_st * e_lo)
            u = dxo - dxt
            v = dyo - dyt
            a = u + to * v
            b = v - to * u
            dist = 4.0 * (a * a * e_lo + b * b * e_so) * r_o
            det2 = (dlo - dlt) + (dso - dst)
            kld = (trace + dist) * 0.5 + det2 - 1.0
            pos = lab == 1
            acc = acc + jnp.where(pos, kld, 0.0)
            cnt = cnt + jnp.where(pos, 1.0, 0.0)
            return acc, cnt

        return lax.fori_loop(0, _GROUPS, group, carry)

    zero = jnp.zeros((_L,), jnp.float32)
    carry = (zero, zero)
    start(0, 0)
    for ci in range(_NCHUNK):
        if ci + 1 < _NCHUNK:
            start(ci + 1, (ci + 1) % _NBUF)
        drain(ci % _NBUF)
        carry = compute(ci % _NBUF, carry)
    acc, cnt = carry
    res_v[0] = acc
    res_v[1] = cnt
    pltpu.sync_copy(res_v, out_hbm.at[wid])


@jax.jit
def _loss(out_ellipse, labels, ellipse_targets):
    def planes_flat(x):
        t = jnp.transpose(x, (2, 0, 1)).reshape(5, 4, 8, 96, 128)
        return jnp.transpose(t, (0, 1, 3, 2, 4)).reshape(-1)

    oe_flat = planes_flat(out_ellipse)
    et_flat = planes_flat(ellipse_targets)
    lab_flat = jnp.transpose(
        labels.reshape(4, 8, 96, 128), (0, 2, 1, 3)).reshape(-1)

    mesh = plsc.VectorSubcoreMesh(core_axis_name="c", subcore_axis_name="s")
    parts = pl.kernel(
        _tile_body,
        mesh=mesh,
        compiler_params=pltpu.CompilerParams(
            needs_layout_passes=False, use_tc_tiling_on_sc=False),
        out_type=jax.ShapeDtypeStruct((_NW, 2, _L), jnp.float32),
        scratch_types=[
            pltpu.VMEM((_NBUF, 5 * _CHUNK), jnp.float32),
            pltpu.VMEM((_NBUF, 5 * _CHUNK), jnp.float32),
            pltpu.VMEM((_NBUF, _CHUNK), jnp.int32),
            pltpu.VMEM((2, _L), jnp.float32),
            pltpu.SemaphoreType.DMA((_NBUF,)),
        ],
    )(oe_flat, et_flat, lab_flat)
    total = jnp.sum(parts[:, 0, :])
    npos = jnp.sum(parts[:, 1, :])
    return total / jnp.maximum(npos, 1.0)


def kernel(out_ellipse, labels, ellipse_targets, anchors):
    return _loss(out_ellipse, labels, ellipse_targets)
